# Initial kernel scaffold; baseline (speedup 1.0000x reference)
import jax
import jax.numpy as jnp
from jax.experimental import pallas as pl


def kernel(logits, step, sample):
    raise NotImplementedError("write your pallas kernel here")


# speedup vs baseline: 8.9121x; 8.9121x over previous
import jax
import jax.numpy as jnp
from jax import lax
from jax.experimental import pallas as pl
from jax.experimental.pallas import tpu as pltpu, tpu_sc as plsc

SIZE = 16777216
GROUP_SIZE = 4
TEMP_INIT = 1.0
TEMP_FINAL = 0.1
ANNEAL_STEPS = 10000

CHUNK = 16384
LANES = 16


def _temperature(step):
    step_f = jnp.maximum(jnp.asarray(step), 0).astype(jnp.float32)
    frac = jnp.minimum(jnp.float32(1.0), step_f / jnp.float32(ANNEAL_STEPS))
    t = jnp.float32(TEMP_INIT) + frac * (jnp.float32(TEMP_FINAL) - jnp.float32(TEMP_INIT))
    return jnp.maximum(t, jnp.float32(1e-06))


def _compute_chunk(in_v, out_v, tv):
    gi = 4 * lax.iota(jnp.int32, LANES)
    one = jnp.full((LANES,), 1.0, dtype=jnp.float32)
    zero = jnp.full((LANES,), 0.0, dtype=jnp.float32)

    def maj(a, b, c):
        return (a & b) | (c & (a | b))

    def blk(i, carry):
        base = gi + i * (4 * LANES)
        v0 = plsc.load_gather(in_v, [base]) / tv
        v1 = plsc.load_gather(in_v, [base + 1]) / tv
        v2 = plsc.load_gather(in_v, [base + 2]) / tv
        v3 = plsc.load_gather(in_v, [base + 3]) / tv
        b01 = v0 >= v1
        b02 = v0 >= v2
        b03 = v0 >= v3
        b12 = v1 >= v2
        b13 = v1 >= v3
        b23 = v2 >= v3
        m0 = maj(b01, b02, b03)
        m1 = maj(~b01, b12, b13)
        m2 = maj(~b02, ~b12, b23)
        m3 = maj(~b03, ~b13, ~b23)
        plsc.store_scatter(out_v, [base], jnp.where(m0, one, zero))
        plsc.store_scatter(out_v, [base + 1], jnp.where(m1, one, zero))
        plsc.store_scatter(out_v, [base + 2], jnp.where(m2, one, zero))
        plsc.store_scatter(out_v, [base + 3], jnp.where(m3, one, zero))
        return carry

    lax.fori_loop(0, CHUNK // (4 * LANES), blk, 0, unroll=4)


def _sc_topk_mask(flat, tvec):
    info = plsc.get_sparse_core_info()
    nc, ns = info.num_cores, info.num_subcores
    nw = nc * ns
    per_w = SIZE // nw
    nch = per_w // CHUNK
    pairs = nch // 2
    mesh = plsc.VectorSubcoreMesh(core_axis_name="c", subcore_axis_name="s")

    def body(x_hbm, t_hbm, out_hbm, in0, in1, out0, out1, t_v,
             sem_i0, sem_i1, sem_o0, sem_o1):
        wid = lax.axis_index("s") * nc + lax.axis_index("c")
        base = wid * per_w
        pltpu.sync_copy(t_hbm, t_v)
        tv = t_v[...]

        def in_slice(g):
            return x_hbm.at[pl.ds(base + g * CHUNK, CHUNK)]

        def out_slice(g):
            return out_hbm.at[pl.ds(base + g * CHUNK, CHUNK)]

        pltpu.async_copy(in_slice(0), in0, sem_i0)
        pltpu.async_copy(in_slice(1), in1, sem_i1)

        pltpu.make_async_copy(in_slice(0), in0, sem_i0).wait()
        _compute_chunk(in0, out0, tv)
        pltpu.async_copy(out0, out_slice(0), sem_o0)
        pltpu.async_copy(in_slice(2), in0, sem_i0)
        pltpu.make_async_copy(in_slice(1), in1, sem_i1).wait()
        _compute_chunk(in1, out1, tv)
        pltpu.async_copy(out1, out_slice(1), sem_o1)
        pltpu.async_copy(in_slice(3), in1, sem_i1)

        def pair(j, carry):
            g0 = 2 * j
            pltpu.make_async_copy(in_slice(g0), in0, sem_i0).wait()
            pltpu.make_async_copy(out0, out_slice(g0), sem_o0).wait()
            _compute_chunk(in0, out0, tv)
            pltpu.async_copy(out0, out_slice(g0), sem_o0)
            pltpu.async_copy(in_slice(g0 + 2), in0, sem_i0)
            pltpu.make_async_copy(in_slice(g0 + 1), in1, sem_i1).wait()
            pltpu.make_async_copy(out1, out_slice(g0 + 1), sem_o1).wait()
            _compute_chunk(in1, out1, tv)
            pltpu.async_copy(out1, out_slice(g0 + 1), sem_o1)
            pltpu.async_copy(in_slice(g0 + 3), in1, sem_i1)
            return carry

        lax.fori_loop(1, pairs - 1, pair, 0)

        g0 = 2 * (pairs - 1)
        pltpu.make_async_copy(in_slice(g0), in0, sem_i0).wait()
        pltpu.make_async_copy(out0, out_slice(g0), sem_o0).wait()
        _compute_chunk(in0, out0, tv)
        pltpu.async_copy(out0, out_slice(g0), sem_o0)
        pltpu.make_async_copy(in_slice(g0 + 1), in1, sem_i1).wait()
        pltpu.make_async_copy(out1, out_slice(g0 + 1), sem_o1).wait()
        _compute_chunk(in1, out1, tv)
        pltpu.async_copy(out1, out_slice(g0 + 1), sem_o1)
        pltpu.make_async_copy(out0, out_slice(g0), sem_o0).wait()
        pltpu.make_async_copy(out1, out_slice(g0 + 1), sem_o1).wait()

    call = pl.kernel(
        body,
        out_type=jax.ShapeDtypeStruct((SIZE,), jnp.float32),
        mesh=mesh,
        compiler_params=pltpu.CompilerParams(needs_layout_passes=False),
        scratch_types=[
            pltpu.VMEM((CHUNK,), jnp.float32),
            pltpu.VMEM((CHUNK,), jnp.float32),
            pltpu.VMEM((CHUNK,), jnp.float32),
            pltpu.VMEM((CHUNK,), jnp.float32),
            pltpu.VMEM((LANES,), jnp.float32),
            pltpu.SemaphoreType.DMA,
            pltpu.SemaphoreType.DMA,
            pltpu.SemaphoreType.DMA,
            pltpu.SemaphoreType.DMA,
        ],
    )
    return call(flat, tvec)


def kernel(logits, step=0, sample=0):
    flat = logits.reshape(-1)
    tvec = jnp.full((LANES,), _temperature(step), dtype=jnp.float32)
    return _sc_topk_mask(flat, tvec)


# speedup vs baseline: 377.2623x; 42.3317x over previous
import jax
import jax.numpy as jnp
from jax import lax
from jax.experimental import pallas as pl
from jax.experimental.pallas import tpu as pltpu, tpu_sc as plsc

SIZE = 16777216
GROUP_SIZE = 4
TEMP_INIT = 1.0
TEMP_FINAL = 0.1
ANNEAL_STEPS = 10000

CHUNK = 16384
TILE = 512
LANES = 16


def _temperature(step):
    step_f = jnp.maximum(jnp.asarray(step), 0).astype(jnp.float32)
    frac = jnp.minimum(jnp.float32(1.0), step_f / jnp.float32(ANNEAL_STEPS))
    t = jnp.float32(TEMP_INIT) + frac * (jnp.float32(TEMP_FINAL) - jnp.float32(TEMP_INIT))
    return jnp.maximum(t, jnp.float32(1e-06))


def _compute_chunk(in_v, out_v, tv):
    qi = 4 * lax.iota(jnp.int32, LANES)
    one = jnp.full((LANES,), 1.0, dtype=jnp.float32)
    zero = jnp.full((LANES,), 0.0, dtype=jnp.float32)

    def maj(a, b, c):
        return (a & b) | (c & (a | b))

    def blk(i, carry):
        toff = (i // 8) * TILE
        goff = (i % 8) * LANES
        off = toff + goff
        v0 = in_v[pl.ds(off, LANES)] / tv
        v1 = in_v[pl.ds(off + 128, LANES)] / tv
        v2 = in_v[pl.ds(off + 256, LANES)] / tv
        v3 = in_v[pl.ds(off + 384, LANES)] / tv
        b01 = v0 >= v1
        b02 = v0 >= v2
        b03 = v0 >= v3
        b12 = v1 >= v2
        b13 = v1 >= v3
        b23 = v2 >= v3
        m0 = maj(b01, b02, b03)
        m1 = maj(~b01, b12, b13)
        m2 = maj(~b02, ~b12, b23)
        m3 = maj(~b03, ~b13, ~b23)
        obase = toff + 4 * goff
        plsc.store_scatter(out_v, [qi + obase], jnp.where(m0, one, zero))
        plsc.store_scatter(out_v, [qi + (obase + 1)], jnp.where(m1, one, zero))
        plsc.store_scatter(out_v, [qi + (obase + 2)], jnp.where(m2, one, zero))
        plsc.store_scatter(out_v, [qi + (obase + 3)], jnp.where(m3, one, zero))
        return carry

    lax.fori_loop(0, CHUNK // (4 * LANES), blk, 0, unroll=4)


def _sc_topk_mask(flat, tvec):
    info = plsc.get_sparse_core_info()
    nc, ns = info.num_cores, info.num_subcores
    nw = nc * ns
    per_w = SIZE // nw
    nch = per_w // CHUNK
    pairs = nch // 2
    mesh = plsc.VectorSubcoreMesh(core_axis_name="c", subcore_axis_name="s")

    def body(x_hbm, t_hbm, out_hbm, in0, in1, out0, out1, t_v,
             sem_i0, sem_i1, sem_o0, sem_o1):
        wid = lax.axis_index("s") * nc + lax.axis_index("c")
        base = wid * per_w
        pltpu.sync_copy(t_hbm, t_v)
        tv = t_v[...]

        def in_slice(g):
            return x_hbm.at[pl.ds(base + g * CHUNK, CHUNK)]

        def out_slice(g):
            return out_hbm.at[pl.ds(base + g * CHUNK, CHUNK)]

        pltpu.async_copy(in_slice(0), in0, sem_i0)
        pltpu.async_copy(in_slice(1), in1, sem_i1)

        pltpu.make_async_copy(in_slice(0), in0, sem_i0).wait()
        _compute_chunk(in0, out0, tv)
        pltpu.async_copy(out0, out_slice(0), sem_o0)
        pltpu.async_copy(in_slice(2), in0, sem_i0)
        pltpu.make_async_copy(in_slice(1), in1, sem_i1).wait()
        _compute_chunk(in1, out1, tv)
        pltpu.async_copy(out1, out_slice(1), sem_o1)
        pltpu.async_copy(in_slice(3), in1, sem_i1)

        def pair(j, carry):
            g0 = 2 * j
            pltpu.make_async_copy(in_slice(g0), in0, sem_i0).wait()
            pltpu.make_async_copy(out0, out_slice(g0), sem_o0).wait()
            _compute_chunk(in0, out0, tv)
            pltpu.async_copy(out0, out_slice(g0), sem_o0)
            pltpu.async_copy(in_slice(g0 + 2), in0, sem_i0)
            pltpu.make_async_copy(in_slice(g0 + 1), in1, sem_i1).wait()
            pltpu.make_async_copy(out1, out_slice(g0 + 1), sem_o1).wait()
            _compute_chunk(in1, out1, tv)
            pltpu.async_copy(out1, out_slice(g0 + 1), sem_o1)
            pltpu.async_copy(in_slice(g0 + 3), in1, sem_i1)
            return carry

        lax.fori_loop(1, pairs - 1, pair, 0)

        g0 = 2 * (pairs - 1)
        pltpu.make_async_copy(in_slice(g0), in0, sem_i0).wait()
        pltpu.make_async_copy(out0, out_slice(g0), sem_o0).wait()
        _compute_chunk(in0, out0, tv)
        pltpu.async_copy(out0, out_slice(g0), sem_o0)
        pltpu.make_async_copy(in_slice(g0 + 1), in1, sem_i1).wait()
        pltpu.make_async_copy(out1, out_slice(g0 + 1), sem_o1).wait()
        _compute_chunk(in1, out1, tv)
        pltpu.async_copy(out1, out_slice(g0 + 1), sem_o1)
        pltpu.make_async_copy(out0, out_slice(g0), sem_o0).wait()
        pltpu.make_async_copy(out1, out_slice(g0 + 1), sem_o1).wait()

    call = pl.kernel(
        body,
        out_type=jax.ShapeDtypeStruct((SIZE,), jnp.float32),
        mesh=mesh,
        compiler_params=pltpu.CompilerParams(needs_layout_passes=False),
        scratch_types=[
            pltpu.VMEM((CHUNK,), jnp.float32),
            pltpu.VMEM((CHUNK,), jnp.float32),
            pltpu.VMEM((CHUNK,), jnp.float32),
            pltpu.VMEM((CHUNK,), jnp.float32),
            pltpu.VMEM((LANES,), jnp.float32),
            pltpu.SemaphoreType.DMA,
            pltpu.SemaphoreType.DMA,
            pltpu.SemaphoreType.DMA,
            pltpu.SemaphoreType.DMA,
        ],
    )
    return call(flat, tvec)


def kernel(logits, step=0, sample=0):
    x = logits.reshape(SIZE // (4 * 128), 128, 4)
    x = jnp.swapaxes(x, 1, 2)
    flat = x.reshape(-1)
    tvec = jnp.full((LANES,), _temperature(step), dtype=jnp.float32)
    return _sc_topk_mask(flat, tvec)


# speedup vs baseline: 479.0879x; 1.2699x over previous
import jax
import jax.numpy as jnp
from jax import lax
from jax.experimental import pallas as pl
from jax.experimental.pallas import tpu as pltpu, tpu_sc as plsc

SIZE = 16777216
GROUP_SIZE = 4
TEMP_INIT = 1.0
TEMP_FINAL = 0.1
ANNEAL_STEPS = 10000

CHUNK = 16384
TILE = 512
LANES = 16


def _temperature(step):
    step_f = jnp.maximum(jnp.asarray(step), 0).astype(jnp.float32)
    frac = jnp.minimum(jnp.float32(1.0), step_f / jnp.float32(ANNEAL_STEPS))
    t = jnp.float32(TEMP_INIT) + frac * (jnp.float32(TEMP_FINAL) - jnp.float32(TEMP_INIT))
    return jnp.maximum(t, jnp.float32(1e-06))


def _compute_chunk(in_v, out_v, tv):
    qi = 4 * lax.iota(jnp.int32, LANES)
    one = jnp.full((LANES,), 1.0, dtype=jnp.float32)
    zero = jnp.full((LANES,), 0.0, dtype=jnp.float32)

    def maj(a, b, c):
        return (a & b) | (c & (a | b))

    @plsc.parallel_loop(0, CHUNK // (4 * LANES), unroll=8)
    def blk(i):
        off = (i // 8) * TILE + (i % 8) * LANES
        v0 = in_v[pl.ds(off, LANES)] / tv
        v1 = in_v[pl.ds(off + 128, LANES)] / tv
        v2 = in_v[pl.ds(off + 256, LANES)] / tv
        v3 = in_v[pl.ds(off + 384, LANES)] / tv
        b01 = v0 >= v1
        b02 = v0 >= v2
        b03 = v0 >= v3
        b12 = v1 >= v2
        b13 = v1 >= v3
        b23 = v2 >= v3
        m0 = maj(b01, b02, b03)
        m1 = maj(~b01, b12, b13)
        m2 = maj(~b02, ~b12, b23)
        m3 = ~maj(b03, b13, b23)
        obase = i * 64
        plsc.store_scatter(out_v, [qi + obase], jnp.where(m0, one, zero))
        plsc.store_scatter(out_v, [qi + (obase + 1)], jnp.where(m1, one, zero))
        plsc.store_scatter(out_v, [qi + (obase + 2)], jnp.where(m2, one, zero))
        plsc.store_scatter(out_v, [qi + (obase + 3)], jnp.where(m3, one, zero))


def _sc_topk_mask(flat, tvec):
    info = plsc.get_sparse_core_info()
    nc, ns = info.num_cores, info.num_subcores
    nw = nc * ns
    per_w = SIZE // nw
    nch = per_w // CHUNK
    pairs = nch // 2
    mesh = plsc.VectorSubcoreMesh(core_axis_name="c", subcore_axis_name="s")

    def body(x_hbm, t_hbm, out_hbm, in0, in1, out0, out1, t_v,
             sem_i0, sem_i1, sem_o0, sem_o1):
        wid = lax.axis_index("s") * nc + lax.axis_index("c")
        base = wid * per_w
        pltpu.sync_copy(t_hbm, t_v)
        tv = t_v[...]

        def in_slice(g):
            return x_hbm.at[pl.ds(base + g * CHUNK, CHUNK)]

        def out_slice(g):
            return out_hbm.at[pl.ds(base + g * CHUNK, CHUNK)]

        pltpu.async_copy(in_slice(0), in0, sem_i0)
        pltpu.async_copy(in_slice(1), in1, sem_i1)

        pltpu.make_async_copy(in_slice(0), in0, sem_i0).wait()
        _compute_chunk(in0, out0, tv)
        pltpu.async_copy(out0, out_slice(0), sem_o0)
        pltpu.async_copy(in_slice(2), in0, sem_i0)
        pltpu.make_async_copy(in_slice(1), in1, sem_i1).wait()
        _compute_chunk(in1, out1, tv)
        pltpu.async_copy(out1, out_slice(1), sem_o1)
        pltpu.async_copy(in_slice(3), in1, sem_i1)

        def pair(j, carry):
            g0 = 2 * j
            pltpu.make_async_copy(in_slice(g0), in0, sem_i0).wait()
            pltpu.make_async_copy(out0, out_slice(g0), sem_o0).wait()
            _compute_chunk(in0, out0, tv)
            pltpu.async_copy(out0, out_slice(g0), sem_o0)
            pltpu.async_copy(in_slice(g0 + 2), in0, sem_i0)
            pltpu.make_async_copy(in_slice(g0 + 1), in1, sem_i1).wait()
            pltpu.make_async_copy(out1, out_slice(g0 + 1), sem_o1).wait()
            _compute_chunk(in1, out1, tv)
            pltpu.async_copy(out1, out_slice(g0 + 1), sem_o1)
            pltpu.async_copy(in_slice(g0 + 3), in1, sem_i1)
            return carry

        lax.fori_loop(1, pairs - 1, pair, 0)

        g0 = 2 * (pairs - 1)
        pltpu.make_async_copy(in_slice(g0), in0, sem_i0).wait()
        pltpu.make_async_copy(out0, out_slice(g0), sem_o0).wait()
        _compute_chunk(in0, out0, tv)
        pltpu.async_copy(out0, out_slice(g0), sem_o0)
        pltpu.make_async_copy(in_slice(g0 + 1), in1, sem_i1).wait()
        pltpu.make_async_copy(out1, out_slice(g0 + 1), sem_o1).wait()
        _compute_chunk(in1, out1, tv)
        pltpu.async_copy(out1, out_slice(g0 + 1), sem_o1)
        pltpu.make_async_copy(out0, out_slice(g0), sem_o0).wait()
        pltpu.make_async_copy(out1, out_slice(g0 + 1), sem_o1).wait()

    call = pl.kernel(
        body,
        out_type=jax.ShapeDtypeStruct((SIZE,), jnp.float32),
        mesh=mesh,
        compiler_params=pltpu.CompilerParams(needs_layout_passes=False),
        scratch_types=[
            pltpu.VMEM((CHUNK,), jnp.float32),
            pltpu.VMEM((CHUNK,), jnp.float32),
            pltpu.VMEM((CHUNK,), jnp.float32),
            pltpu.VMEM((CHUNK,), jnp.float32),
            pltpu.VMEM((LANES,), jnp.float32),
            pltpu.SemaphoreType.DMA,
            pltpu.SemaphoreType.DMA,
            pltpu.SemaphoreType.DMA,
            pltpu.SemaphoreType.DMA,
        ],
    )
    return call(flat, tvec)


def kernel(logits, step=0, sample=0):
    x = logits.reshape(SIZE // (4 * 128), 128, 4)
    x = jnp.swapaxes(x, 1, 2)
    flat = x.reshape(-1)
    tvec = jnp.full((LANES,), _temperature(step), dtype=jnp.float32)
    return _sc_topk_mask(flat, tvec)


# speedup vs baseline: 569.7186x; 1.1892x over previous
import jax
import jax.numpy as jnp
from jax import lax
from jax.experimental import pallas as pl
from jax.experimental.pallas import tpu as pltpu, tpu_sc as plsc

SIZE = 16777216
GROUP_SIZE = 4
TEMP_INIT = 1.0
TEMP_FINAL = 0.1
ANNEAL_STEPS = 10000

CHUNK = 16384
TILE = 512
LANES = 16


def _temperature(step):
    step_f = jnp.maximum(jnp.asarray(step), 0).astype(jnp.float32)
    frac = jnp.minimum(jnp.float32(1.0), step_f / jnp.float32(ANNEAL_STEPS))
    t = jnp.float32(TEMP_INIT) + frac * (jnp.float32(TEMP_FINAL) - jnp.float32(TEMP_INIT))
    return jnp.maximum(t, jnp.float32(1e-06))


def _compute_chunk(in_v, out_v, tv):
    qi = 4 * lax.iota(jnp.int32, LANES)
    one = jnp.full((LANES,), 1.0, dtype=jnp.float32)
    zero = jnp.full((LANES,), 0.0, dtype=jnp.float32)

    def maj(a, b, c):
        return (a & b) | (c & (a | b))

    step64 = jnp.full((LANES,), 64, dtype=jnp.int32)
    carry0 = (qi, qi + 1, qi + 2, qi + 3)

    @plsc.parallel_loop(0, CHUNK // (4 * LANES), unroll=8, carry=carry0)
    def blk(i, c):
        i0, i1, i2, i3 = c
        off = (i // 8) * TILE + (i % 8) * LANES
        v0 = in_v[pl.ds(off, LANES)] / tv
        v1 = in_v[pl.ds(off + 128, LANES)] / tv
        v2 = in_v[pl.ds(off + 256, LANES)] / tv
        v3 = in_v[pl.ds(off + 384, LANES)] / tv
        b01 = v0 >= v1
        b02 = v0 >= v2
        b03 = v0 >= v3
        b12 = v1 >= v2
        b13 = v1 >= v3
        b23 = v2 >= v3
        m0 = maj(b01, b02, b03)
        m1 = maj(~b01, b12, b13)
        m2 = maj(~b02, ~b12, b23)
        m3 = ~maj(b03, b13, b23)
        plsc.store_scatter(out_v, [i0], jnp.where(m0, one, zero))
        plsc.store_scatter(out_v, [i1], jnp.where(m1, one, zero))
        plsc.store_scatter(out_v, [i2], jnp.where(m2, one, zero))
        plsc.store_scatter(out_v, [i3], jnp.where(m3, one, zero))
        return (i0 + step64, i1 + step64, i2 + step64, i3 + step64)


def _sc_topk_mask(flat, tvec):
    info = plsc.get_sparse_core_info()
    nc, ns = info.num_cores, info.num_subcores
    nw = nc * ns
    per_w = SIZE // nw
    nch = per_w // CHUNK
    pairs = nch // 2
    mesh = plsc.VectorSubcoreMesh(core_axis_name="c", subcore_axis_name="s")

    def body(x_hbm, t_hbm, out_hbm, in0, in1, out0, out1, t_v,
             sem_i0, sem_i1, sem_o0, sem_o1):
        wid = lax.axis_index("s") * nc + lax.axis_index("c")
        base = wid * per_w
        pltpu.sync_copy(t_hbm, t_v)
        tv = t_v[...]

        def in_slice(g):
            return x_hbm.at[pl.ds(base + g * CHUNK, CHUNK)]

        def out_slice(g):
            return out_hbm.at[pl.ds(base + g * CHUNK, CHUNK)]

        pltpu.async_copy(in_slice(0), in0, sem_i0)
        pltpu.async_copy(in_slice(1), in1, sem_i1)

        pltpu.make_async_copy(in_slice(0), in0, sem_i0).wait()
        _compute_chunk(in0, out0, tv)
        pltpu.async_copy(out0, out_slice(0), sem_o0)
        pltpu.async_copy(in_slice(2), in0, sem_i0)
        pltpu.make_async_copy(in_slice(1), in1, sem_i1).wait()
        _compute_chunk(in1, out1, tv)
        pltpu.async_copy(out1, out_slice(1), sem_o1)
        pltpu.async_copy(in_slice(3), in1, sem_i1)

        def pair(j, carry):
            g0 = 2 * j
            pltpu.make_async_copy(in_slice(g0), in0, sem_i0).wait()
            pltpu.make_async_copy(out0, out_slice(g0), sem_o0).wait()
            _compute_chunk(in0, out0, tv)
            pltpu.async_copy(out0, out_slice(g0), sem_o0)
            pltpu.async_copy(in_slice(g0 + 2), in0, sem_i0)
            pltpu.make_async_copy(in_slice(g0 + 1), in1, sem_i1).wait()
            pltpu.make_async_copy(out1, out_slice(g0 + 1), sem_o1).wait()
            _compute_chunk(in1, out1, tv)
            pltpu.async_copy(out1, out_slice(g0 + 1), sem_o1)
            pltpu.async_copy(in_slice(g0 + 3), in1, sem_i1)
            return carry

        lax.fori_loop(1, pairs - 1, pair, 0)

        g0 = 2 * (pairs - 1)
        pltpu.make_async_copy(in_slice(g0), in0, sem_i0).wait()
        pltpu.make_async_copy(out0, out_slice(g0), sem_o0).wait()
        _compute_chunk(in0, out0, tv)
        pltpu.async_copy(out0, out_slice(g0), sem_o0)
        pltpu.make_async_copy(in_slice(g0 + 1), in1, sem_i1).wait()
        pltpu.make_async_copy(out1, out_slice(g0 + 1), sem_o1).wait()
        _compute_chunk(in1, out1, tv)
        pltpu.async_copy(out1, out_slice(g0 + 1), sem_o1)
        pltpu.make_async_copy(out0, out_slice(g0), sem_o0).wait()
        pltpu.make_async_copy(out1, out_slice(g0 + 1), sem_o1).wait()

    call = pl.kernel(
        body,
        out_type=jax.ShapeDtypeStruct((SIZE,), jnp.float32),
        mesh=mesh,
        compiler_params=pltpu.CompilerParams(needs_layout_passes=False),
        scratch_types=[
            pltpu.VMEM((CHUNK,), jnp.float32),
            pltpu.VMEM((CHUNK,), jnp.float32),
            pltpu.VMEM((CHUNK,), jnp.float32),
            pltpu.VMEM((CHUNK,), jnp.float32),
            pltpu.VMEM((LANES,), jnp.float32),
            pltpu.SemaphoreType.DMA,
            pltpu.SemaphoreType.DMA,
            pltpu.SemaphoreType.DMA,
            pltpu.SemaphoreType.DMA,
        ],
    )
    return call(flat, tvec)


def kernel(logits, step=0, sample=0):
    x = logits.reshape(SIZE // (4 * 128), 128, 4)
    x = jnp.swapaxes(x, 1, 2)
    flat = x.reshape(-1)
    tvec = jnp.full((LANES,), _temperature(step), dtype=jnp.float32)
    return _sc_topk_mask(flat, tvec)
